# baseline (speedup 1.0000x reference)
import jax
import jax.numpy as jnp
from jax import lax
from jax.experimental import pallas as pl
from jax.experimental.pallas import tpu as pltpu
from jax.experimental.pallas import tpu_sc as plsc

N = 10000
D = 128
E = 320000
NP = 10240
NA = NP + 8
NW = 32
EPW = E // NW
KC = 128
NCH = 80
NSB = 5
BR = 640
NSL = NP // 16
QS = 1.0 / 8388608.0


def _chunk_rows(ch):
    return 3 * lax.bitwise_and(ch, 31)


def _sb_load(edges_hbm, sblk, wid, f):
    half = lax.bitwise_and(f, 1)
    pltpu.sync_copy(edges_hbm.at[wid, f], sblk.at[pl.ds(half * 48, 48)])


def _deg_body(edges_hbm, deg_hbm, lw_hbm, zbuf, sblk, ones0, ones1, lidx0,
              lidx1, ewf0, ewf1, degs, loops, semA0, semA1, semB0, semB1):
    cid = lax.axis_index("c")
    sid = lax.axis_index("s")
    wid = cid * 16 + sid

    def zbody(i, _):
        zbuf[pl.ds(i * 16, 16)] = jnp.zeros((16,), jnp.float32)
        return 0

    lax.fori_loop(0, NSL // 16, zbody, 0)
    pltpu.sync_copy(zbuf, degs.at[pl.ds(sid * NSL, NSL)])

    def obody(i, _):
        zbuf[pl.ds(i * 16, 16)] = jnp.full((16,), 1.0, jnp.float32)
        return 0

    lax.fori_loop(0, NSL // 16, obody, 0)
    pltpu.sync_copy(zbuf, loops.at[pl.ds(sid * NSL, NSL)])

    @pl.when(sid == 0)
    def _():
        pltpu.sync_copy(zbuf.at[pl.ds(0, 8)], degs.at[pl.ds(NP, 8)])
        pltpu.sync_copy(zbuf.at[pl.ds(0, 8)], loops.at[pl.ds(NP, 8)])

    plsc.subcore_barrier()

    one = jnp.full((16,), 1.0, jnp.float32)
    zero = jnp.zeros((16,), jnp.float32)
    trash = jnp.full((16,), NP, jnp.int32)

    _sb_load(edges_hbm, sblk, wid, 0)

    def compute(ch, onesv, lidxv, ewf):
        base = _chunk_rows(ch)

        def group(g, cntv):
            r = sblk[base, pl.ds(g * 16, 16)]
            c = sblk[base + 1, pl.ds(g * 16, 16)]
            q = sblk[base + 2, pl.ds(g * 16, 16)]
            is_loop = r == c
            ones = jnp.where(is_loop, zero, one)
            onesv[pl.ds(g * 16, 16)] = ones
            lidxv[pl.ds(g * 16, 16)] = jnp.where(is_loop, c, trash)
            ewf[pl.ds(g * 16, 16)] = q.astype(jnp.float32) * QS
            return cntv + (one - ones)

        cntv = lax.fori_loop(0, KC // 16, group, zero)
        cnt = cntv[0]
        for k in range(1, 16):
            cnt = cnt + cntv[k]
        return cnt

    def step(i, _):
        @pl.when(jnp.logical_and(lax.bitwise_and(i, 7) == 0,
                                 lax.shift_right_logical(i, 3) + 1 < NSB))
        def _():
            _sb_load(edges_hbm, sblk, wid, lax.shift_right_logical(i, 3) + 1)

        a = 2 * i
        b = 2 * i + 1
        cnt0 = compute(a, ones0, lidx0, ewf0)
        pltpu.async_copy(ones0, degs.at[sblk.at[_chunk_rows(a) + 1]], semA0,
                         add=True)

        @pl.when(cnt0 > 0.0)
        def _():
            pltpu.async_copy(ewf0, loops.at[lidx0], semB0)

        cnt1 = compute(b, ones1, lidx1, ewf1)
        pltpu.async_copy(ones1, degs.at[sblk.at[_chunk_rows(b) + 1]], semA1,
                         add=True)

        @pl.when(cnt1 > 0.0)
        def _():
            pltpu.async_copy(ewf1, loops.at[lidx1], semB1)

        pltpu.make_async_copy(ones0, degs.at[sblk.at[_chunk_rows(a) + 1]],
                              semA0).wait()

        @pl.when(cnt0 > 0.0)
        def _():
            pltpu.make_async_copy(ewf0, loops.at[lidx0], semB0).wait()

        pltpu.make_async_copy(ones1, degs.at[sblk.at[_chunk_rows(b) + 1]],
                              semA1).wait()

        @pl.when(cnt1 > 0.0)
        def _():
            pltpu.make_async_copy(ewf1, loops.at[lidx1], semB1).wait()

        return 0

    lax.fori_loop(0, NCH // 2, step, 0)

    plsc.subcore_barrier()
    pltpu.sync_copy(degs.at[pl.ds(sid * NSL, NSL)],
                    deg_hbm.at[cid, pl.ds(sid * NSL, NSL)])
    pltpu.sync_copy(loops.at[pl.ds(sid * NSL, NSL)],
                    lw_hbm.at[cid, pl.ds(sid * NSL, NSL)])


def _deg_loopw_partials(edges):
    mesh = plsc.VectorSubcoreMesh(core_axis_name="c", subcore_axis_name="s")
    k = pl.kernel(
        _deg_body,
        out_type=(
            jax.ShapeDtypeStruct((2, NP), jnp.float32),
            jax.ShapeDtypeStruct((2, NP), jnp.float32),
        ),
        mesh=mesh,
        scratch_types=[
            pltpu.VMEM((NSL,), jnp.float32),
            pltpu.VMEM((96, KC), jnp.int32),
            pltpu.VMEM((KC,), jnp.float32),
            pltpu.VMEM((KC,), jnp.float32),
            pltpu.VMEM((KC,), jnp.int32),
            pltpu.VMEM((KC,), jnp.int32),
            pltpu.VMEM((KC,), jnp.float32),
            pltpu.VMEM((KC,), jnp.float32),
            pltpu.VMEM_SHARED((NA,), jnp.float32),
            pltpu.VMEM_SHARED((NA,), jnp.float32),
            pltpu.SemaphoreType.DMA,
            pltpu.SemaphoreType.DMA,
            pltpu.SemaphoreType.DMA,
            pltpu.SemaphoreType.DMA,
        ],
    )
    return k(edges)


def _msg_body(edges_hbm, yp_hbm, out_hbm, idx0, idx1, colv0, colv1, buf0,
              buf1, acc, rowsem0, rowsem1, scatsem0, scatsem1, ldsem0,
              ldsem1):
    cid = lax.axis_index("c")
    sid = lax.axis_index("s")
    wid = cid * 16 + sid

    def zrow(i, _):
        for j in range(D // 16):
            buf0[i, pl.ds(j * 16, 16)] = jnp.zeros((16,), jnp.float32)
        return 0

    lax.fori_loop(0, KC, zrow, 0)
    for t in range(NSL // KC):
        pltpu.sync_copy(buf0, acc.at[pl.ds(sid * NSL + t * KC, KC)])

    @pl.when(wid == 0)
    def _():
        pltpu.sync_copy(buf0.at[pl.ds(0, 8)], acc.at[pl.ds(NP, 8)])

    zero = jnp.zeros((16,), jnp.float32)

    def scale(idxb, colv, buf):
        def sgroup(g, _):
            r = idxb[0, pl.ds(g * 16, 16)]
            c = idxb[1, pl.ds(g * 16, 16)]
            q = idxb[2, pl.ds(g * 16, 16)]
            colv[pl.ds(g * 16, 16)] = c
            w = jnp.where(r == c, zero, q.astype(jnp.float32) * QS)
            for lane in range(16):
                s = w[lane]
                e = g * 16 + lane
                for j in range(D // 16):
                    buf[e, pl.ds(j * 16, 16)] = buf[e, pl.ds(j * 16, 16)] * s
            return 0

        lax.fori_loop(0, KC // 16, sgroup, 0)

    pltpu.sync_copy(edges_hbm.at[wid, 0], idx0)
    pltpu.async_copy(yp_hbm.at[idx0.at[0]], buf0, rowsem0)
    pltpu.sync_copy(edges_hbm.at[wid, 1], idx1)
    pltpu.async_copy(yp_hbm.at[idx1.at[0]], buf1, rowsem1)
    plsc.subcore_barrier()

    def step(i, _):
        a = 2 * i
        b = 2 * i + 1
        pltpu.make_async_copy(yp_hbm.at[idx0.at[0]], buf0, rowsem0).wait()
        scale(idx0, colv0, buf0)
        pltpu.async_copy(buf0, acc.at[idx0.at[1]], scatsem0, add=True)
        pltpu.make_async_copy(yp_hbm.at[idx1.at[0]], buf1, rowsem1).wait()
        scale(idx1, colv1, buf1)
        pltpu.async_copy(buf1, acc.at[idx1.at[1]], scatsem1, add=True)
        pltpu.make_async_copy(buf0, acc.at[idx0.at[1]], scatsem0).wait()

        @pl.when(a + 2 < NCH)
        def _():
            pltpu.sync_copy(edges_hbm.at[wid, a + 2], idx0)
            pltpu.async_copy(yp_hbm.at[idx0.at[0]], buf0, rowsem0)

        pltpu.make_async_copy(buf1, acc.at[idx1.at[1]], scatsem1).wait()

        @pl.when(b + 2 < NCH)
        def _():
            pltpu.sync_copy(edges_hbm.at[wid, b + 2], idx1)
            pltpu.async_copy(yp_hbm.at[idx1.at[0]], buf1, rowsem1)

        return 0

    lax.fori_loop(0, NCH // 2, step, 0)

    plsc.subcore_barrier()
    for t in range(NSL // KC):
        off = sid * NSL + t * KC
        pltpu.sync_copy(acc.at[pl.ds(off, KC)], out_hbm.at[cid, pl.ds(off, KC)])


def _msg_partials(edges, yp):
    mesh = plsc.VectorSubcoreMesh(core_axis_name="c", subcore_axis_name="s")
    k = pl.kernel(
        _msg_body,
        out_type=jax.ShapeDtypeStruct((2, NP, D), jnp.float32),
        mesh=mesh,
        scratch_types=[
            pltpu.VMEM((3, KC), jnp.int32),
            pltpu.VMEM((3, KC), jnp.int32),
            pltpu.VMEM((KC,), jnp.int32),
            pltpu.VMEM((KC,), jnp.int32),
            pltpu.VMEM((KC, D), jnp.float32),
            pltpu.VMEM((KC, D), jnp.float32),
            pltpu.VMEM_SHARED((NA, D), jnp.float32),
            pltpu.SemaphoreType.DMA,
            pltpu.SemaphoreType.DMA,
            pltpu.SemaphoreType.DMA,
            pltpu.SemaphoreType.DMA,
            pltpu.SemaphoreType.DMA,
            pltpu.SemaphoreType.DMA,
        ],
    )
    return k(edges, yp)


def _first_body(degp_ref, lwp_ref, x_ref, w_ref, b_ref, yp_ref, dis_ref,
                lw_ref):
    deg = jnp.sum(degp_ref[...], axis=0) + 1.0
    dis = lax.rsqrt(deg)[:, None]
    lw0 = lwp_ref[0]
    lw1 = lwp_ref[1]
    lw_ref[...] = jnp.where(lw0 == 1.0, lw1, lw0)[:, None]
    y = jnp.dot(x_ref[...], w_ref[...], preferred_element_type=jnp.float32)
    yp_ref[...] = (y + b_ref[...]) * dis
    dis_ref[...] = dis


def _first_stage(degp, lwp, x, wt, b):
    return pl.pallas_call(
        _first_body,
        grid=(NP // BR,),
        in_specs=[
            pl.BlockSpec((2, BR), lambda i: (0, i)),
            pl.BlockSpec((2, BR), lambda i: (0, i)),
            pl.BlockSpec((BR, D), lambda i: (i, 0)),
            pl.BlockSpec((D, D), lambda i: (0, 0)),
            pl.BlockSpec((1, D), lambda i: (0, 0)),
        ],
        out_specs=[
            pl.BlockSpec((BR, D), lambda i: (i, 0)),
            pl.BlockSpec((BR, 1), lambda i: (i, 0)),
            pl.BlockSpec((BR, 1), lambda i: (i, 0)),
        ],
        out_shape=[
            jax.ShapeDtypeStruct((NP, D), jnp.float32),
            jax.ShapeDtypeStruct((NP, 1), jnp.float32),
            jax.ShapeDtypeStruct((NP, 1), jnp.float32),
        ],
    )(degp, lwp, x, wt, b)


def _mid_body(sp_ref, yp_ref, lw_ref, dis_ref, w_ref, b_ref, out_ref):
    s = sp_ref[0] + sp_ref[1] + lw_ref[...] * yp_ref[...]
    h = jnp.maximum(s * dis_ref[...], 0.0)
    y = jnp.dot(h, w_ref[...], preferred_element_type=jnp.float32)
    out_ref[...] = (y + b_ref[...]) * dis_ref[...]


def _mid_stage(sp, yp, lw, dis, wt, b):
    return pl.pallas_call(
        _mid_body,
        grid=(NP // BR,),
        in_specs=[
            pl.BlockSpec((2, BR, D), lambda i: (0, i, 0)),
            pl.BlockSpec((BR, D), lambda i: (i, 0)),
            pl.BlockSpec((BR, 1), lambda i: (i, 0)),
            pl.BlockSpec((BR, 1), lambda i: (i, 0)),
            pl.BlockSpec((D, D), lambda i: (0, 0)),
            pl.BlockSpec((1, D), lambda i: (0, 0)),
        ],
        out_specs=pl.BlockSpec((BR, D), lambda i: (i, 0)),
        out_shape=jax.ShapeDtypeStruct((NP, D), jnp.float32),
    )(sp, yp, lw, dis, wt, b)


def _final_body(sp_ref, yp_ref, lw_ref, dis_ref, out_ref):
    s = sp_ref[0] + sp_ref[1] + lw_ref[...] * yp_ref[...]
    out_ref[...] = jax.nn.sigmoid(s * dis_ref[...])


def _final_stage(sp, yp, lw, dis):
    return pl.pallas_call(
        _final_body,
        grid=(NP // BR,),
        in_specs=[
            pl.BlockSpec((2, BR, D), lambda i: (0, i, 0)),
            pl.BlockSpec((BR, D), lambda i: (i, 0)),
            pl.BlockSpec((BR, 1), lambda i: (i, 0)),
            pl.BlockSpec((BR, 1), lambda i: (i, 0)),
        ],
        out_specs=pl.BlockSpec((BR, D), lambda i: (i, 0)),
        out_shape=jax.ShapeDtypeStruct((NP, D), jnp.float32),
    )(sp, yp, lw, dis)


@jax.jit
def kernel(x, edge_index, edge_weight, W0, b0, W1, b1):
    pad = NCH * KC - EPW
    rw = jnp.pad(edge_index[0].reshape(NW, EPW), ((0, 0), (0, pad)))
    cw = jnp.pad(edge_index[1].reshape(NW, EPW), ((0, 0), (0, pad)),
                 constant_values=NP)
    qw = jnp.pad(
        (edge_weight[:, 0] * 8388608.0 + 0.5).astype(jnp.int32).reshape(
            NW, EPW), ((0, 0), (0, pad)))
    edges = jnp.stack(
        [rw.reshape(NW, NCH, KC), cw.reshape(NW, NCH, KC),
         qw.reshape(NW, NCH, KC)], axis=2)
    edges_sb = edges.reshape(NW, NSB, 48, KC)

    xp = jnp.zeros((NP, D), jnp.float32).at[:N].set(x)

    degp, lwp = _deg_loopw_partials(edges_sb)
    yp0, dis, lw = _first_stage(degp, lwp, xp, W0.T, b0[None, :])
    sp0 = _msg_partials(edges, yp0)
    yp1 = _mid_stage(sp0, yp0, lw, dis, W1.T, b1[None, :])
    sp1 = _msg_partials(edges, yp1)
    out = _final_stage(sp1, yp1, lw, dis)
    return out[:N]


# speedup vs baseline: 1.0028x; 1.0028x over previous
import jax
import jax.numpy as jnp
from jax import lax
from jax.experimental import pallas as pl
from jax.experimental.pallas import tpu as pltpu
from jax.experimental.pallas import tpu_sc as plsc

N = 10000
D = 128
E = 320000
NP = 10240
NA = NP + 128
NW = 32
EPW = E // NW
KC = 128
NCH = 80
NSB = 5
BR = 640
NSL = NP // 16
QS = 1.0 / 8388608.0


def _chunk_rows(ch):
    return 3 * lax.bitwise_and(ch, 31)


def _sb_load(edges_hbm, sblk, wid, f):
    half = lax.bitwise_and(f, 1)
    pltpu.sync_copy(edges_hbm.at[wid, f], sblk.at[pl.ds(half * 48, 48)])


def _deg_body(edges_hbm, deg_hbm, lw_hbm, zbuf, sblk, ones0, ones1, lidx0,
              lidx1, ewf0, ewf1, degs, loops, semA0, semA1, semB0, semB1):
    cid = lax.axis_index("c")
    sid = lax.axis_index("s")
    wid = cid * 16 + sid

    def zbody(i, _):
        zbuf[pl.ds(i * 16, 16)] = jnp.zeros((16,), jnp.float32)
        return 0

    lax.fori_loop(0, NSL // 16, zbody, 0)
    pltpu.sync_copy(zbuf, degs.at[pl.ds(sid * NSL, NSL)])

    def obody(i, _):
        zbuf[pl.ds(i * 16, 16)] = jnp.full((16,), 1.0, jnp.float32)
        return 0

    lax.fori_loop(0, NSL // 16, obody, 0)
    pltpu.sync_copy(zbuf, loops.at[pl.ds(sid * NSL, NSL)])

    plsc.subcore_barrier()

    one = jnp.full((16,), 1.0, jnp.float32)
    zero = jnp.zeros((16,), jnp.float32)
    trash = jnp.full((16,), NP, jnp.int32)

    _sb_load(edges_hbm, sblk, wid, 0)

    def compute(ch, onesv, lidxv, ewf):
        base = _chunk_rows(ch)

        def group(g, cntv):
            r = sblk[base, pl.ds(g * 16, 16)]
            c = sblk[base + 1, pl.ds(g * 16, 16)]
            q = sblk[base + 2, pl.ds(g * 16, 16)]
            is_loop = r == c
            ones = jnp.where(is_loop, zero, one)
            onesv[pl.ds(g * 16, 16)] = ones
            lidxv[pl.ds(g * 16, 16)] = jnp.where(is_loop, c, trash)
            ewf[pl.ds(g * 16, 16)] = q.astype(jnp.float32) * QS
            return cntv + (one - ones)

        cntv = lax.fori_loop(0, KC // 16, group, zero)
        cnt = cntv[0]
        for k in range(1, 16):
            cnt = cnt + cntv[k]
        return cnt

    def step(i, _):
        @pl.when(jnp.logical_and(lax.bitwise_and(i, 7) == 0,
                                 lax.shift_right_logical(i, 3) + 1 < NSB))
        def _():
            _sb_load(edges_hbm, sblk, wid, lax.shift_right_logical(i, 3) + 1)

        a = 2 * i
        b = 2 * i + 1
        cnt0 = compute(a, ones0, lidx0, ewf0)
        pltpu.async_copy(ones0, degs.at[sblk.at[_chunk_rows(a) + 1]], semA0,
                         add=True)

        @pl.when(cnt0 > 0.0)
        def _():
            pltpu.async_copy(ewf0, loops.at[lidx0], semB0)

        cnt1 = compute(b, ones1, lidx1, ewf1)
        pltpu.async_copy(ones1, degs.at[sblk.at[_chunk_rows(b) + 1]], semA1,
                         add=True)

        @pl.when(cnt1 > 0.0)
        def _():
            pltpu.async_copy(ewf1, loops.at[lidx1], semB1)

        pltpu.make_async_copy(ones0, degs.at[sblk.at[_chunk_rows(a) + 1]],
                              semA0).wait()

        @pl.when(cnt0 > 0.0)
        def _():
            pltpu.make_async_copy(ewf0, loops.at[lidx0], semB0).wait()

        pltpu.make_async_copy(ones1, degs.at[sblk.at[_chunk_rows(b) + 1]],
                              semA1).wait()

        @pl.when(cnt1 > 0.0)
        def _():
            pltpu.make_async_copy(ewf1, loops.at[lidx1], semB1).wait()

        return 0

    lax.fori_loop(0, NCH // 2, step, 0)

    plsc.subcore_barrier()
    pltpu.sync_copy(degs.at[pl.ds(sid * NSL, NSL)],
                    deg_hbm.at[cid, pl.ds(sid * NSL, NSL)])
    pltpu.sync_copy(loops.at[pl.ds(sid * NSL, NSL)],
                    lw_hbm.at[cid, pl.ds(sid * NSL, NSL)])


def _deg_loopw_partials(edges):
    mesh = plsc.VectorSubcoreMesh(core_axis_name="c", subcore_axis_name="s")
    k = pl.kernel(
        _deg_body,
        out_type=(
            jax.ShapeDtypeStruct((2, NP), jnp.float32),
            jax.ShapeDtypeStruct((2, NP), jnp.float32),
        ),
        mesh=mesh,
        scratch_types=[
            pltpu.VMEM((NSL,), jnp.float32),
            pltpu.VMEM((96, KC), jnp.int32),
            pltpu.VMEM((KC,), jnp.float32),
            pltpu.VMEM((KC,), jnp.float32),
            pltpu.VMEM((KC,), jnp.int32),
            pltpu.VMEM((KC,), jnp.int32),
            pltpu.VMEM((KC,), jnp.float32),
            pltpu.VMEM((KC,), jnp.float32),
            pltpu.VMEM_SHARED((NA,), jnp.float32),
            pltpu.VMEM_SHARED((NA,), jnp.float32),
            pltpu.SemaphoreType.DMA,
            pltpu.SemaphoreType.DMA,
            pltpu.SemaphoreType.DMA,
            pltpu.SemaphoreType.DMA,
        ],
    )
    return k(edges)


def _msg_body(edges_hbm, yp_hbm, out_hbm, idx0, idx1, colv0, colv1, buf0,
              buf1, acc, rowsem0, rowsem1, scatsem0, scatsem1, ldsem0,
              ldsem1):
    cid = lax.axis_index("c")
    sid = lax.axis_index("s")
    wid = cid * 16 + sid

    def zrow(i, _):
        for j in range(D // 16):
            buf0[i, pl.ds(j * 16, 16)] = jnp.zeros((16,), jnp.float32)
        return 0

    lax.fori_loop(0, KC, zrow, 0)
    for t in range(NSL // KC):
        pltpu.sync_copy(buf0, acc.at[pl.ds(sid * NSL + t * KC, KC)])

    zero = jnp.zeros((16,), jnp.float32)

    def scale(idxb, colv, buf):
        def sgroup(g, _):
            r = idxb[0, pl.ds(g * 16, 16)]
            c = idxb[1, pl.ds(g * 16, 16)]
            q = idxb[2, pl.ds(g * 16, 16)]
            colv[pl.ds(g * 16, 16)] = c
            w = jnp.where(r == c, zero, q.astype(jnp.float32) * QS)
            for lane in range(16):
                s = w[lane]
                e = g * 16 + lane
                for j in range(D // 16):
                    buf[e, pl.ds(j * 16, 16)] = buf[e, pl.ds(j * 16, 16)] * s
            return 0

        lax.fori_loop(0, KC // 16, sgroup, 0)

    pltpu.sync_copy(edges_hbm.at[wid, 0], idx0)
    pltpu.async_copy(yp_hbm.at[idx0.at[0]], buf0, rowsem0)
    pltpu.sync_copy(edges_hbm.at[wid, 1], idx1)
    pltpu.async_copy(yp_hbm.at[idx1.at[0]], buf1, rowsem1)
    plsc.subcore_barrier()

    def step(i, _):
        a = 2 * i
        b = 2 * i + 1
        pltpu.make_async_copy(yp_hbm.at[idx0.at[0]], buf0, rowsem0).wait()
        scale(idx0, colv0, buf0)
        pltpu.async_copy(buf0, acc.at[idx0.at[1]], scatsem0, add=True)
        pltpu.make_async_copy(yp_hbm.at[idx1.at[0]], buf1, rowsem1).wait()
        scale(idx1, colv1, buf1)
        pltpu.async_copy(buf1, acc.at[idx1.at[1]], scatsem1, add=True)
        pltpu.make_async_copy(buf0, acc.at[idx0.at[1]], scatsem0).wait()

        @pl.when(a + 2 < NCH)
        def _():
            pltpu.sync_copy(edges_hbm.at[wid, a + 2], idx0)
            pltpu.async_copy(yp_hbm.at[idx0.at[0]], buf0, rowsem0)

        pltpu.make_async_copy(buf1, acc.at[idx1.at[1]], scatsem1).wait()

        @pl.when(b + 2 < NCH)
        def _():
            pltpu.sync_copy(edges_hbm.at[wid, b + 2], idx1)
            pltpu.async_copy(yp_hbm.at[idx1.at[0]], buf1, rowsem1)

        return 0

    lax.fori_loop(0, NCH // 2, step, 0)

    plsc.subcore_barrier()
    for t in range(NSL // KC):
        off = sid * NSL + t * KC
        pltpu.sync_copy(acc.at[pl.ds(off, KC)], out_hbm.at[cid, pl.ds(off, KC)])


def _msg_partials(edges, yp):
    mesh = plsc.VectorSubcoreMesh(core_axis_name="c", subcore_axis_name="s")
    k = pl.kernel(
        _msg_body,
        out_type=jax.ShapeDtypeStruct((2, NP, D), jnp.float32),
        mesh=mesh,
        scratch_types=[
            pltpu.VMEM((3, KC), jnp.int32),
            pltpu.VMEM((3, KC), jnp.int32),
            pltpu.VMEM((KC,), jnp.int32),
            pltpu.VMEM((KC,), jnp.int32),
            pltpu.VMEM((KC, D), jnp.float32),
            pltpu.VMEM((KC, D), jnp.float32),
            pltpu.VMEM_SHARED((NA, D), jnp.float32),
            pltpu.SemaphoreType.DMA,
            pltpu.SemaphoreType.DMA,
            pltpu.SemaphoreType.DMA,
            pltpu.SemaphoreType.DMA,
            pltpu.SemaphoreType.DMA,
            pltpu.SemaphoreType.DMA,
        ],
    )
    return k(edges, yp)


def _first_body(degp_ref, lwp_ref, x_ref, w_ref, b_ref, yp_ref, dis_ref,
                lw_ref):
    deg = jnp.sum(degp_ref[...], axis=0) + 1.0
    dis = lax.rsqrt(deg)[:, None]
    lw0 = lwp_ref[0]
    lw1 = lwp_ref[1]
    lw_ref[...] = jnp.where(lw0 == 1.0, lw1, lw0)[:, None]
    y = jnp.dot(x_ref[...], w_ref[...], preferred_element_type=jnp.float32)
    yp_ref[...] = (y + b_ref[...]) * dis
    dis_ref[...] = dis


def _first_stage(degp, lwp, x, wt, b):
    return pl.pallas_call(
        _first_body,
        grid=(NP // BR,),
        in_specs=[
            pl.BlockSpec((2, BR), lambda i: (0, i)),
            pl.BlockSpec((2, BR), lambda i: (0, i)),
            pl.BlockSpec((BR, D), lambda i: (i, 0)),
            pl.BlockSpec((D, D), lambda i: (0, 0)),
            pl.BlockSpec((1, D), lambda i: (0, 0)),
        ],
        out_specs=[
            pl.BlockSpec((BR, D), lambda i: (i, 0)),
            pl.BlockSpec((BR, 1), lambda i: (i, 0)),
            pl.BlockSpec((BR, 1), lambda i: (i, 0)),
        ],
        out_shape=[
            jax.ShapeDtypeStruct((NP, D), jnp.float32),
            jax.ShapeDtypeStruct((NP, 1), jnp.float32),
            jax.ShapeDtypeStruct((NP, 1), jnp.float32),
        ],
    )(degp, lwp, x, wt, b)


def _mid_body(sp_ref, yp_ref, lw_ref, dis_ref, w_ref, b_ref, out_ref):
    s = sp_ref[0] + sp_ref[1] + lw_ref[...] * yp_ref[...]
    h = jnp.maximum(s * dis_ref[...], 0.0)
    y = jnp.dot(h, w_ref[...], preferred_element_type=jnp.float32)
    out_ref[...] = (y + b_ref[...]) * dis_ref[...]


def _mid_stage(sp, yp, lw, dis, wt, b):
    return pl.pallas_call(
        _mid_body,
        grid=(NP // BR,),
        in_specs=[
            pl.BlockSpec((2, BR, D), lambda i: (0, i, 0)),
            pl.BlockSpec((BR, D), lambda i: (i, 0)),
            pl.BlockSpec((BR, 1), lambda i: (i, 0)),
            pl.BlockSpec((BR, 1), lambda i: (i, 0)),
            pl.BlockSpec((D, D), lambda i: (0, 0)),
            pl.BlockSpec((1, D), lambda i: (0, 0)),
        ],
        out_specs=pl.BlockSpec((BR, D), lambda i: (i, 0)),
        out_shape=jax.ShapeDtypeStruct((NP, D), jnp.float32),
    )(sp, yp, lw, dis, wt, b)


def _final_body(sp_ref, yp_ref, lw_ref, dis_ref, out_ref):
    s = sp_ref[0] + sp_ref[1] + lw_ref[...] * yp_ref[...]
    out_ref[...] = jax.nn.sigmoid(s * dis_ref[...])


def _final_stage(sp, yp, lw, dis):
    return pl.pallas_call(
        _final_body,
        grid=(NP // BR,),
        in_specs=[
            pl.BlockSpec((2, BR, D), lambda i: (0, i, 0)),
            pl.BlockSpec((BR, D), lambda i: (i, 0)),
            pl.BlockSpec((BR, 1), lambda i: (i, 0)),
            pl.BlockSpec((BR, 1), lambda i: (i, 0)),
        ],
        out_specs=pl.BlockSpec((BR, D), lambda i: (i, 0)),
        out_shape=jax.ShapeDtypeStruct((NP, D), jnp.float32),
    )(sp, yp, lw, dis)


@jax.jit
def kernel(x, edge_index, edge_weight, W0, b0, W1, b1):
    pad = NCH * KC - EPW
    rw = jnp.pad(edge_index[0].reshape(NW, EPW), ((0, 0), (0, pad)))
    cw = jnp.concatenate(
        [edge_index[1].reshape(NW, EPW),
         jnp.broadcast_to(NP + (jnp.arange(pad, dtype=jnp.int32) % 128),
                          (NW, pad))], axis=1)
    qw = jnp.pad(
        (edge_weight[:, 0] * 8388608.0 + 0.5).astype(jnp.int32).reshape(
            NW, EPW), ((0, 0), (0, pad)))
    edges = jnp.stack(
        [rw.reshape(NW, NCH, KC), cw.reshape(NW, NCH, KC),
         qw.reshape(NW, NCH, KC)], axis=2)
    edges_sb = edges.reshape(NW, NSB, 48, KC)

    xp = jnp.zeros((NP, D), jnp.float32).at[:N].set(x)

    degp, lwp = _deg_loopw_partials(edges_sb)
    yp0, dis, lw = _first_stage(degp, lwp, xp, W0.T, b0[None, :])
    sp0 = _msg_partials(edges, yp0)
    yp1 = _mid_stage(sp0, yp0, lw, dis, W1.T, b1[None, :])
    sp1 = _msg_partials(edges, yp1)
    out = _final_stage(sp1, yp1, lw, dis)
    return out[:N]


# speedup vs baseline: 1.4584x; 1.4544x over previous
import jax
import jax.numpy as jnp
from jax import lax
from jax.experimental import pallas as pl
from jax.experimental.pallas import tpu as pltpu
from jax.experimental.pallas import tpu_sc as plsc

N = 10000
D = 128
E = 320000
NP = 10240
NA = NP + 128
NW = 32
EPW = E // NW
KC = 128
NCH = 80
NSB = 5
BR = 640
NSL = NP // 16
QS = 1.0 / 8388608.0


def _chunk_rows(ch):
    return 3 * lax.bitwise_and(ch, 31)


def _sb_load(edges_hbm, sblk, wid, f):
    half = lax.bitwise_and(f, 1)
    pltpu.sync_copy(edges_hbm.at[wid, f], sblk.at[pl.ds(half * 48, 48)])


def _deg_body(edges_hbm, deg_hbm, lw_hbm, zbuf, sblk, ones0, ones1, lidx0,
              lidx1, ewf0, ewf1, degs, loops, semA0, semA1, semB0, semB1):
    cid = lax.axis_index("c")
    sid = lax.axis_index("s")
    wid = cid * 16 + sid

    def zbody(i, _):
        zbuf[pl.ds(i * 16, 16)] = jnp.zeros((16,), jnp.float32)
        return 0

    lax.fori_loop(0, NSL // 16, zbody, 0)
    pltpu.sync_copy(zbuf, degs.at[pl.ds(sid * NSL, NSL)])

    def obody(i, _):
        zbuf[pl.ds(i * 16, 16)] = jnp.full((16,), 1.0, jnp.float32)
        return 0

    lax.fori_loop(0, NSL // 16, obody, 0)
    pltpu.sync_copy(zbuf, loops.at[pl.ds(sid * NSL, NSL)])

    plsc.subcore_barrier()

    one = jnp.full((16,), 1.0, jnp.float32)
    zero = jnp.zeros((16,), jnp.float32)
    trash = jnp.full((16,), NP, jnp.int32)

    _sb_load(edges_hbm, sblk, wid, 0)

    def compute(ch, onesv, lidxv, ewf):
        base = _chunk_rows(ch)

        def group(g, cntv):
            r = sblk[base, pl.ds(g * 16, 16)]
            c = sblk[base + 1, pl.ds(g * 16, 16)]
            q = sblk[base + 2, pl.ds(g * 16, 16)]
            is_loop = r == c
            ones = jnp.where(is_loop, zero, one)
            onesv[pl.ds(g * 16, 16)] = ones
            lidxv[pl.ds(g * 16, 16)] = jnp.where(is_loop, c, trash)
            ewf[pl.ds(g * 16, 16)] = q.astype(jnp.float32) * QS
            return cntv + (one - ones)

        cntv = lax.fori_loop(0, KC // 16, group, zero)
        cnt = cntv[0]
        for k in range(1, 16):
            cnt = cnt + cntv[k]
        return cnt

    def step(i, _):
        @pl.when(jnp.logical_and(lax.bitwise_and(i, 7) == 0,
                                 lax.shift_right_logical(i, 3) + 1 < NSB))
        def _():
            _sb_load(edges_hbm, sblk, wid, lax.shift_right_logical(i, 3) + 1)

        a = 2 * i
        b = 2 * i + 1
        cnt0 = compute(a, ones0, lidx0, ewf0)
        pltpu.async_copy(ones0, degs.at[sblk.at[_chunk_rows(a) + 1]], semA0,
                         add=True)

        @pl.when(cnt0 > 0.0)
        def _():
            pltpu.async_copy(ewf0, loops.at[lidx0], semB0)

        cnt1 = compute(b, ones1, lidx1, ewf1)
        pltpu.async_copy(ones1, degs.at[sblk.at[_chunk_rows(b) + 1]], semA1,
                         add=True)

        @pl.when(cnt1 > 0.0)
        def _():
            pltpu.async_copy(ewf1, loops.at[lidx1], semB1)

        pltpu.make_async_copy(ones0, degs.at[sblk.at[_chunk_rows(a) + 1]],
                              semA0).wait()

        @pl.when(cnt0 > 0.0)
        def _():
            pltpu.make_async_copy(ewf0, loops.at[lidx0], semB0).wait()

        pltpu.make_async_copy(ones1, degs.at[sblk.at[_chunk_rows(b) + 1]],
                              semA1).wait()

        @pl.when(cnt1 > 0.0)
        def _():
            pltpu.make_async_copy(ewf1, loops.at[lidx1], semB1).wait()

        return 0

    lax.fori_loop(0, NCH // 2, step, 0)

    plsc.subcore_barrier()
    pltpu.sync_copy(degs.at[pl.ds(sid * NSL, NSL)],
                    deg_hbm.at[cid, pl.ds(sid * NSL, NSL)])
    pltpu.sync_copy(loops.at[pl.ds(sid * NSL, NSL)],
                    lw_hbm.at[cid, pl.ds(sid * NSL, NSL)])


def _deg_loopw_partials(edges):
    mesh = plsc.VectorSubcoreMesh(core_axis_name="c", subcore_axis_name="s")
    k = pl.kernel(
        _deg_body,
        out_type=(
            jax.ShapeDtypeStruct((2, NP), jnp.float32),
            jax.ShapeDtypeStruct((2, NP), jnp.float32),
        ),
        mesh=mesh,
        scratch_types=[
            pltpu.VMEM((NSL,), jnp.float32),
            pltpu.VMEM((96, KC), jnp.int32),
            pltpu.VMEM((KC,), jnp.float32),
            pltpu.VMEM((KC,), jnp.float32),
            pltpu.VMEM((KC,), jnp.int32),
            pltpu.VMEM((KC,), jnp.int32),
            pltpu.VMEM((KC,), jnp.float32),
            pltpu.VMEM((KC,), jnp.float32),
            pltpu.VMEM_SHARED((NA,), jnp.float32),
            pltpu.VMEM_SHARED((NA,), jnp.float32),
            pltpu.SemaphoreType.DMA,
            pltpu.SemaphoreType.DMA,
            pltpu.SemaphoreType.DMA,
            pltpu.SemaphoreType.DMA,
        ],
    )
    return k(edges)


NCHM = 79


def _msg_body(edges_hbm, ews_hbm, yp_hbm, out_hbm, idx0, idx1, ewv0, ewv1,
              buf0, buf1, acc, rowsem0, rowsem1, scatsem0, scatsem1, ldsem):
    cid = lax.axis_index("c")
    sid = lax.axis_index("s")
    wid = cid * 16 + sid

    def zrow(i, _):
        for j in range(D // 16):
            buf0[i, pl.ds(j * 16, 16)] = jnp.zeros((16,), jnp.float32)
        return 0

    lax.fori_loop(0, KC, zrow, 0)
    for t in range(NSL // KC):
        pltpu.sync_copy(buf0, acc.at[pl.ds(sid * NSL + t * KC, KC)])

    zero = jnp.zeros((16,), jnp.float32)

    def scale(idxb, ewv, buf):
        def sgroup(g, _):
            r = idxb[0, pl.ds(g * 16, 16)]
            c = idxb[1, pl.ds(g * 16, 16)]
            w = jnp.where(r == c, zero, ewv[pl.ds(g * 16, 16)])
            for lane in range(16):
                s = w[lane]
                e = g * 16 + lane
                for j in range(D // 16):
                    buf[e, pl.ds(j * 16, 16)] = buf[e, pl.ds(j * 16, 16)] * s
            return 0

        lax.fori_loop(0, KC // 16, sgroup, 0)

    def load(ch, idxb, ewv):
        pltpu.async_copy(edges_hbm.at[wid, ch], idxb, ldsem)
        pltpu.async_copy(ews_hbm.at[wid, ch], ewv, ldsem)
        pltpu.make_async_copy(edges_hbm.at[wid, ch], idxb, ldsem).wait()
        pltpu.make_async_copy(ews_hbm.at[wid, ch], ewv, ldsem).wait()

    load(0, idx0, ewv0)
    pltpu.async_copy(yp_hbm.at[idx0.at[0]], buf0, rowsem0)
    load(1, idx1, ewv1)
    pltpu.async_copy(yp_hbm.at[idx1.at[0]], buf1, rowsem1)
    plsc.subcore_barrier()

    def step(i, _):
        pltpu.make_async_copy(yp_hbm.at[idx0.at[0]], buf0, rowsem0).wait()
        scale(idx0, ewv0, buf0)
        pltpu.async_copy(buf0, acc.at[idx0.at[1]], scatsem0, add=True)
        pltpu.make_async_copy(yp_hbm.at[idx1.at[0]], buf1, rowsem1).wait()
        scale(idx1, ewv1, buf1)
        pltpu.async_copy(buf1, acc.at[idx1.at[1]], scatsem1, add=True)
        pltpu.make_async_copy(buf0, acc.at[idx0.at[1]], scatsem0).wait()
        load(2 * i + 2, idx0, ewv0)
        pltpu.async_copy(yp_hbm.at[idx0.at[0]], buf0, rowsem0)
        pltpu.make_async_copy(buf1, acc.at[idx1.at[1]], scatsem1).wait()

        @pl.when(2 * i + 3 < NCHM)
        def _():
            load(2 * i + 3, idx1, ewv1)
            pltpu.async_copy(yp_hbm.at[idx1.at[0]], buf1, rowsem1)

        return 0

    lax.fori_loop(0, (NCHM - 1) // 2, step, 0)
    pltpu.make_async_copy(yp_hbm.at[idx0.at[0]], buf0, rowsem0).wait()
    scale(idx0, ewv0, buf0)
    pltpu.sync_copy(buf0, acc.at[idx0.at[1]], add=True)

    plsc.subcore_barrier()
    for t in range(NSL // KC):
        off = sid * NSL + t * KC
        pltpu.sync_copy(acc.at[pl.ds(off, KC)], out_hbm.at[cid, pl.ds(off, KC)])


def _msg_partials(edges, ews, yp):
    mesh = plsc.VectorSubcoreMesh(core_axis_name="c", subcore_axis_name="s")
    k = pl.kernel(
        _msg_body,
        out_type=jax.ShapeDtypeStruct((2, NP, D), jnp.float32),
        mesh=mesh,
        scratch_types=[
            pltpu.VMEM((2, KC), jnp.int32),
            pltpu.VMEM((2, KC), jnp.int32),
            pltpu.VMEM((KC,), jnp.float32),
            pltpu.VMEM((KC,), jnp.float32),
            pltpu.VMEM((KC, D), jnp.float32),
            pltpu.VMEM((KC, D), jnp.float32),
            pltpu.VMEM_SHARED((NA, D), jnp.float32),
            pltpu.SemaphoreType.DMA,
            pltpu.SemaphoreType.DMA,
            pltpu.SemaphoreType.DMA,
            pltpu.SemaphoreType.DMA,
            pltpu.SemaphoreType.DMA,
        ],
    )
    return k(edges, ews, yp)


def _first_body(degp_ref, lwp_ref, x_ref, w_ref, b_ref, yp_ref, dis_ref,
                lw_ref):
    deg = jnp.sum(degp_ref[...], axis=0) + 1.0
    dis = lax.rsqrt(deg)[:, None]
    lw0 = lwp_ref[0]
    lw1 = lwp_ref[1]
    lw_ref[...] = jnp.where(lw0 == 1.0, lw1, lw0)[:, None]
    y = jnp.dot(x_ref[...], w_ref[...], preferred_element_type=jnp.float32)
    yp_ref[...] = (y + b_ref[...]) * dis
    dis_ref[...] = dis


def _first_stage(degp, lwp, x, wt, b):
    return pl.pallas_call(
        _first_body,
        grid=(NP // BR,),
        in_specs=[
            pl.BlockSpec((2, BR), lambda i: (0, i)),
            pl.BlockSpec((2, BR), lambda i: (0, i)),
            pl.BlockSpec((BR, D), lambda i: (i, 0)),
            pl.BlockSpec((D, D), lambda i: (0, 0)),
            pl.BlockSpec((1, D), lambda i: (0, 0)),
        ],
        out_specs=[
            pl.BlockSpec((BR, D), lambda i: (i, 0)),
            pl.BlockSpec((BR, 1), lambda i: (i, 0)),
            pl.BlockSpec((BR, 1), lambda i: (i, 0)),
        ],
        out_shape=[
            jax.ShapeDtypeStruct((NP, D), jnp.float32),
            jax.ShapeDtypeStruct((NP, 1), jnp.float32),
            jax.ShapeDtypeStruct((NP, 1), jnp.float32),
        ],
    )(degp, lwp, x, wt, b)


def _mid_body(sp_ref, yp_ref, lw_ref, dis_ref, w_ref, b_ref, out_ref):
    s = sp_ref[0] + sp_ref[1] + lw_ref[...] * yp_ref[...]
    h = jnp.maximum(s * dis_ref[...], 0.0)
    y = jnp.dot(h, w_ref[...], preferred_element_type=jnp.float32)
    out_ref[...] = (y + b_ref[...]) * dis_ref[...]


def _mid_stage(sp, yp, lw, dis, wt, b):
    return pl.pallas_call(
        _mid_body,
        grid=(NP // BR,),
        in_specs=[
            pl.BlockSpec((2, BR, D), lambda i: (0, i, 0)),
            pl.BlockSpec((BR, D), lambda i: (i, 0)),
            pl.BlockSpec((BR, 1), lambda i: (i, 0)),
            pl.BlockSpec((BR, 1), lambda i: (i, 0)),
            pl.BlockSpec((D, D), lambda i: (0, 0)),
            pl.BlockSpec((1, D), lambda i: (0, 0)),
        ],
        out_specs=pl.BlockSpec((BR, D), lambda i: (i, 0)),
        out_shape=jax.ShapeDtypeStruct((NP, D), jnp.float32),
    )(sp, yp, lw, dis, wt, b)


def _final_body(sp_ref, yp_ref, lw_ref, dis_ref, out_ref):
    s = sp_ref[0] + sp_ref[1] + lw_ref[...] * yp_ref[...]
    out_ref[...] = jax.nn.sigmoid(s * dis_ref[...])


def _final_stage(sp, yp, lw, dis):
    return pl.pallas_call(
        _final_body,
        grid=(NP // BR,),
        in_specs=[
            pl.BlockSpec((2, BR, D), lambda i: (0, i, 0)),
            pl.BlockSpec((BR, D), lambda i: (i, 0)),
            pl.BlockSpec((BR, 1), lambda i: (i, 0)),
            pl.BlockSpec((BR, 1), lambda i: (i, 0)),
        ],
        out_specs=pl.BlockSpec((BR, D), lambda i: (i, 0)),
        out_shape=jax.ShapeDtypeStruct((NP, D), jnp.float32),
    )(sp, yp, lw, dis)


@jax.jit
def kernel(x, edge_index, edge_weight, W0, b0, W1, b1):
    row_w = edge_index[0].reshape(NW, EPW)
    col_w = edge_index[1].reshape(NW, EPW)
    ew_w = edge_weight[:, 0].reshape(NW, EPW)
    trash = NP + (jnp.arange(NCH * KC - EPW, dtype=jnp.int32) % 128)

    pad = NCH * KC - EPW
    rw = jnp.pad(row_w, ((0, 0), (0, pad)))
    cw = jnp.concatenate([col_w, jnp.broadcast_to(trash, (NW, pad))], axis=1)
    qw = jnp.pad((ew_w * 8388608.0 + 0.5).astype(jnp.int32),
                 ((0, 0), (0, pad)))
    edges_sb = jnp.stack(
        [rw.reshape(NW, NCH, KC), cw.reshape(NW, NCH, KC),
         qw.reshape(NW, NCH, KC)], axis=2).reshape(NW, NSB, 48, KC)

    padm = NCHM * KC - EPW
    rwm = jnp.pad(row_w, ((0, 0), (0, padm)))
    cwm = jnp.concatenate([col_w, jnp.broadcast_to(trash[:padm], (NW, padm))],
                          axis=1)
    ewm = jnp.pad(ew_w, ((0, 0), (0, padm)))
    edges_m = jnp.stack(
        [rwm.reshape(NW, NCHM, KC), cwm.reshape(NW, NCHM, KC)], axis=2)
    ews_m = ewm.reshape(NW, NCHM, KC)

    xp = jnp.zeros((NP, D), jnp.float32).at[:N].set(x)

    degp, lwp = _deg_loopw_partials(edges_sb)
    yp0, dis, lw = _first_stage(degp, lwp, xp, W0.T, b0[None, :])
    sp0 = _msg_partials(edges_m, ews_m, yp0)
    yp1 = _mid_stage(sp0, yp0, lw, dis, W1.T, b1[None, :])
    sp1 = _msg_partials(edges_m, ews_m, yp1)
    out = _final_stage(sp1, yp1, lw, dis)
    return out[:N]
